# baseline (speedup 1.0000x reference)
import functools

import jax
import jax.numpy as jnp
from jax import lax
from jax.experimental import pallas as pl
from jax.experimental.pallas import tpu as pltpu
from jax.experimental.pallas import tpu_sc as plsc

N = 10000
E = 320000
D = 128

NC = 2
NS = 16
NW = NC * NS
CHUNK = 128
CPT = 160
NSEG = 4
SEG = CPT // NSEG
TOT_CHUNKS = NS * CPT
HCPT = SEG
E_PAD = TOT_CHUNKS * CHUNK
RPT = 632
NROWS = NS * RPT


def _mm_body(h_ref, w_ref, o_ref):
    o_ref[...] = jnp.dot(h_ref[...], w_ref[...],
                         preferred_element_type=jnp.float32)


def _matmul(h, W, rows_per_block=2000):
    return pl.pallas_call(
        _mm_body,
        grid=(N // rows_per_block,),
        in_specs=[
            pl.BlockSpec((rows_per_block, D), lambda i: (i, 0)),
            pl.BlockSpec((D, D), lambda i: (0, 0)),
        ],
        out_specs=pl.BlockSpec((rows_per_block, D), lambda i: (i, 0)),
        out_shape=jax.ShapeDtypeStruct((N, D), jnp.float32),
    )(h, W)


def _mid_body(p_ref, b_ref, w_ref, o_ref):
    h = jnp.maximum(p_ref[...] + b_ref[...], 0.0)
    o_ref[...] = jnp.dot(h, w_ref[...], preferred_element_type=jnp.float32)


def _mid(agg, b, W, rows_per_block=2000):
    return pl.pallas_call(
        _mid_body,
        grid=(N // rows_per_block,),
        in_specs=[
            pl.BlockSpec((rows_per_block, D), lambda i: (i, 0)),
            pl.BlockSpec((1, D), lambda i: (0, 0)),
            pl.BlockSpec((D, D), lambda i: (0, 0)),
        ],
        out_specs=pl.BlockSpec((rows_per_block, D), lambda i: (i, 0)),
        out_shape=jax.ShapeDtypeStruct((N, D), jnp.float32),
    )(agg, b, W)


def _fin_body(p_ref, b_ref, x_ref, o_ref):
    h = jnp.maximum(p_ref[...] + b_ref[...], 0.0)
    o_ref[...] = (h + x_ref[...]) * 0.5


def _fin(agg, b, x, rows_per_block=2000):
    return pl.pallas_call(
        _fin_body,
        grid=(N // rows_per_block,),
        in_specs=[
            pl.BlockSpec((rows_per_block, D), lambda i: (i, 0)),
            pl.BlockSpec((1, D), lambda i: (0, 0)),
            pl.BlockSpec((rows_per_block, D), lambda i: (i, 0)),
        ],
        out_specs=pl.BlockSpec((rows_per_block, D), lambda i: (i, 0)),
        out_shape=jax.ShapeDtypeStruct((N, D), jnp.float32),
    )(agg, b, x)


NBUF = 2


@functools.partial(
    pl.kernel,
    mesh=plsc.VectorSubcoreMesh(core_axis_name="c", subcore_axis_name="s"),
    out_type=jax.ShapeDtypeStruct((NROWS, D), jnp.float32),
    scratch_types=[
        pltpu.VMEM((HCPT, CHUNK), jnp.int32),
        pltpu.VMEM((HCPT, CHUNK), jnp.int32),
        pltpu.VMEM((CHUNK, D), jnp.float32),
        pltpu.VMEM((CHUNK, D), jnp.float32),
        pltpu.VMEM_SHARED((NROWS, D), jnp.float32),
        pltpu.SemaphoreType.DMA,
        pltpu.SemaphoreType.DMA,
        pltpu.SemaphoreType.DMA,
        pltpu.SemaphoreType.DMA,
    ],
)
def _sc_aggregate(src_hbm, dst_hbm, hw_hbm, z_hbm, out_hbm,
                  src_v, dst_v, rows0, rows1, agg_sh,
                  gs0, gs1, ss0, ss1):
    bufs = (rows0, rows1)
    gsems = (gs0, gs1)
    ssems = (ss0, ss1)
    cid = lax.axis_index("c")
    sid = lax.axis_index("s")

    @pl.when(cid == 0)
    def _core0():
        row0 = sid * RPT
        pltpu.sync_copy(z_hbm.at[pl.ds(row0, RPT)],
                        agg_sh.at[pl.ds(row0, RPT)])

        base = sid * CPT

        for h in range(NSEG):
            off = base + h * SEG
            pltpu.sync_copy(src_hbm.at[pl.ds(off, HCPT)], src_v)
            pltpu.sync_copy(dst_hbm.at[pl.ds(off, HCPT)], dst_v)
            if h == 0:
                plsc.subcore_barrier()

            for b in range(NBUF):
                pltpu.async_copy(hw_hbm.at[src_v.at[b]], bufs[b], gsems[b])

            def pair(p, carry):
                for b in range(NBUF):
                    k = p * NBUF + b
                    pltpu.make_async_copy(hw_hbm.at[src_v.at[k]], bufs[b],
                                          gsems[b]).wait()
                    pltpu.async_copy(bufs[b], agg_sh.at[dst_v.at[k]],
                                     ssems[b], add=True)
                for b in range(NBUF):
                    kn = p * NBUF + b + NBUF
                    pltpu.make_async_copy(bufs[b], agg_sh.at[dst_v.at[0]],
                                          ssems[b]).wait()
                    pltpu.async_copy(hw_hbm.at[src_v.at[kn]], bufs[b],
                                     gsems[b])
                return carry

            lax.fori_loop(0, SEG // NBUF - 1, pair, 0)
            for b in range(NBUF):
                k = (SEG // NBUF - 1) * NBUF + b
                pltpu.make_async_copy(hw_hbm.at[src_v.at[k]], bufs[b],
                                      gsems[b]).wait()
                pltpu.async_copy(bufs[b], agg_sh.at[dst_v.at[k]], ssems[b],
                                 add=True)
            for b in range(NBUF):
                pltpu.make_async_copy(bufs[b], agg_sh.at[dst_v.at[0]],
                                      ssems[b]).wait()

        plsc.subcore_barrier()
        pltpu.sync_copy(agg_sh.at[pl.ds(row0, RPT)],
                        out_hbm.at[pl.ds(row0, RPT)])


def kernel(x, edge_index, W1, b1, W2, b2):
    src = edge_index[0].astype(jnp.int32)
    dst = edge_index[1].astype(jnp.int32)
    pad = E_PAD - E
    src2d = jnp.concatenate([src, jnp.zeros((pad,), jnp.int32)]).reshape(
        TOT_CHUNKS, CHUNK)
    dst2d = jnp.concatenate([dst, jnp.full((pad,), N, jnp.int32)]).reshape(
        TOT_CHUNKS, CHUNK)
    zeros = jnp.zeros((NROWS, D), jnp.float32)
    b1r = b1.reshape(1, D)
    b2r = b2.reshape(1, D)

    hw1 = _matmul(x, W1)
    parts1 = _sc_aggregate(src2d, dst2d, hw1, zeros)
    hw2 = _mid(parts1, b1r, W2)
    parts2 = _sc_aggregate(src2d, dst2d, hw2, zeros)
    return _fin(parts2, b2r, x)


# speedup vs baseline: 1.2570x; 1.2570x over previous
import functools

import jax
import jax.numpy as jnp
from jax import lax
from jax.experimental import pallas as pl
from jax.experimental.pallas import tpu as pltpu
from jax.experimental.pallas import tpu_sc as plsc

N = 10000
E = 320000
D = 128

NC = 2
NS = 16
NW = NC * NS
CHUNK = 128
CPT0 = 96
CPT1 = 64
NSEG = 4
HCPT = max(CPT0, CPT1) // NSEG
TOT_CHUNKS = NS * (CPT0 + CPT1)
TOT_PAD = TOT_CHUNKS + 8
E_PAD = TOT_PAD * CHUNK
RPT = 632
NROWS = NS * RPT


def _mm_body(h_ref, w_ref, o_ref):
    o_ref[...] = jnp.dot(h_ref[...], w_ref[...],
                         preferred_element_type=jnp.float32)


def _matmul(h, W, rows_per_block=2000):
    return pl.pallas_call(
        _mm_body,
        grid=(N // rows_per_block,),
        in_specs=[
            pl.BlockSpec((rows_per_block, D), lambda i: (i, 0)),
            pl.BlockSpec((D, D), lambda i: (0, 0)),
        ],
        out_specs=pl.BlockSpec((rows_per_block, D), lambda i: (i, 0)),
        out_shape=jax.ShapeDtypeStruct((N, D), jnp.float32),
    )(h, W)


def _mid_body(p_ref, b_ref, w_ref, o_ref):
    h = jnp.maximum(p_ref[0] + p_ref[1] + b_ref[...], 0.0)
    o_ref[...] = jnp.dot(h, w_ref[...], preferred_element_type=jnp.float32)


def _mid(agg, b, W, rows_per_block=2000):
    return pl.pallas_call(
        _mid_body,
        grid=(N // rows_per_block,),
        in_specs=[
            pl.BlockSpec((2, rows_per_block, D), lambda i: (0, i, 0)),
            pl.BlockSpec((1, D), lambda i: (0, 0)),
            pl.BlockSpec((D, D), lambda i: (0, 0)),
        ],
        out_specs=pl.BlockSpec((rows_per_block, D), lambda i: (i, 0)),
        out_shape=jax.ShapeDtypeStruct((N, D), jnp.float32),
    )(agg, b, W)


def _fin_body(p_ref, b_ref, x_ref, o_ref):
    h = jnp.maximum(p_ref[0] + p_ref[1] + b_ref[...], 0.0)
    o_ref[...] = (h + x_ref[...]) * 0.5


def _fin(agg, b, x, rows_per_block=2000):
    return pl.pallas_call(
        _fin_body,
        grid=(N // rows_per_block,),
        in_specs=[
            pl.BlockSpec((2, rows_per_block, D), lambda i: (0, i, 0)),
            pl.BlockSpec((1, D), lambda i: (0, 0)),
            pl.BlockSpec((rows_per_block, D), lambda i: (i, 0)),
        ],
        out_specs=pl.BlockSpec((rows_per_block, D), lambda i: (i, 0)),
        out_shape=jax.ShapeDtypeStruct((N, D), jnp.float32),
    )(agg, b, x)


NBUF = 2


@functools.partial(
    pl.kernel,
    mesh=plsc.VectorSubcoreMesh(core_axis_name="c", subcore_axis_name="s"),
    out_type=jax.ShapeDtypeStruct((2, NROWS, D), jnp.float32),
    scratch_types=[
        pltpu.VMEM((HCPT, CHUNK), jnp.int32),
        pltpu.VMEM((HCPT, CHUNK), jnp.int32),
        pltpu.VMEM((CHUNK, D), jnp.float32),
        pltpu.VMEM((CHUNK, D), jnp.float32),
        pltpu.VMEM_SHARED((NROWS, D), jnp.float32),
        pltpu.SemaphoreType.DMA,
        pltpu.SemaphoreType.DMA,
        pltpu.SemaphoreType.DMA,
        pltpu.SemaphoreType.DMA,
    ],
)
def _sc_aggregate(src_hbm, dst_hbm, hw_hbm, z_hbm, out_hbm,
                  src_v, dst_v, rows0, rows1, agg_sh,
                  gs0, gs1, ss0, ss1):
    bufs = (rows0, rows1)
    gsems = (gs0, gs1)
    ssems = (ss0, ss1)
    cid = lax.axis_index("c")
    sid = lax.axis_index("s")

    row0 = sid * RPT
    pltpu.sync_copy(z_hbm.at[pl.ds(row0, RPT)], agg_sh.at[pl.ds(row0, RPT)])

    base = jnp.where(cid == 0, sid * CPT0, NS * CPT0 + sid * CPT1)
    seg = jnp.where(cid == 0, CPT0 // NSEG, CPT1 // NSEG)
    spairs = seg // NBUF

    for h in range(NSEG):
        off = base + h * seg
        pltpu.sync_copy(src_hbm.at[pl.ds(off, HCPT)], src_v)
        pltpu.sync_copy(dst_hbm.at[pl.ds(off, HCPT)], dst_v)
        if h == 0:
            plsc.subcore_barrier()

        for b in range(NBUF):
            pltpu.async_copy(hw_hbm.at[src_v.at[b]], bufs[b], gsems[b])

        def pair(p, carry):
            for b in range(NBUF):
                k = p * NBUF + b
                pltpu.make_async_copy(hw_hbm.at[src_v.at[k]], bufs[b],
                                      gsems[b]).wait()
                pltpu.async_copy(bufs[b], agg_sh.at[dst_v.at[k]],
                                 ssems[b], add=True)
            for b in range(NBUF):
                kn = p * NBUF + b + NBUF
                pltpu.make_async_copy(bufs[b], agg_sh.at[dst_v.at[0]],
                                      ssems[b]).wait()
                pltpu.async_copy(hw_hbm.at[src_v.at[kn]], bufs[b],
                                 gsems[b])
            return carry

        lax.fori_loop(0, spairs - 1, pair, 0)
        for b in range(NBUF):
            k = (spairs - 1) * NBUF + b
            pltpu.make_async_copy(hw_hbm.at[src_v.at[k]], bufs[b],
                                  gsems[b]).wait()
            pltpu.async_copy(bufs[b], agg_sh.at[dst_v.at[k]], ssems[b],
                             add=True)
        for b in range(NBUF):
            pltpu.make_async_copy(bufs[b], agg_sh.at[dst_v.at[0]],
                                  ssems[b]).wait()

    plsc.subcore_barrier()
    pltpu.sync_copy(agg_sh.at[pl.ds(row0, RPT)],
                    out_hbm.at[cid, pl.ds(row0, RPT)])


def kernel(x, edge_index, W1, b1, W2, b2):
    src = edge_index[0].astype(jnp.int32)
    dst = edge_index[1].astype(jnp.int32)
    pad = E_PAD - E
    src2d = jnp.concatenate([src, jnp.zeros((pad,), jnp.int32)]).reshape(
        TOT_PAD, CHUNK)
    dst2d = jnp.concatenate([dst, jnp.full((pad,), N, jnp.int32)]).reshape(
        TOT_PAD, CHUNK)
    zeros = jnp.zeros((NROWS, D), jnp.float32)
    b1r = b1.reshape(1, D)
    b2r = b2.reshape(1, D)

    hw1 = _matmul(x, W1)
    parts1 = _sc_aggregate(src2d, dst2d, hw1, zeros)
    hw2 = _mid(parts1, b1r, W2)
    parts2 = _sc_aggregate(src2d, dst2d, hw2, zeros)
    return _fin(parts2, b2r, x)


# speedup vs baseline: 1.3468x; 1.0714x over previous
import functools

import jax
import jax.numpy as jnp
from jax import lax
from jax.experimental import pallas as pl
from jax.experimental.pallas import tpu as pltpu
from jax.experimental.pallas import tpu_sc as plsc

N = 10000
E = 320000
D = 128

NC = 2
NS = 16
NW = NC * NS
CHUNK = 128
CPT0 = 128
CPT1 = 32
NSEG = 4
HCPT = max(CPT0, CPT1) // NSEG
TOT_CHUNKS = NS * (CPT0 + CPT1)
TOT_PAD = TOT_CHUNKS + 8
E_PAD = TOT_PAD * CHUNK
RPT = 632
NROWS = NS * RPT


def _mm_body(h_ref, w_ref, o_ref):
    o_ref[...] = jnp.dot(h_ref[...], w_ref[...],
                         preferred_element_type=jnp.float32)


def _matmul(h, W, rows_per_block=2000):
    return pl.pallas_call(
        _mm_body,
        grid=(N // rows_per_block,),
        in_specs=[
            pl.BlockSpec((rows_per_block, D), lambda i: (i, 0)),
            pl.BlockSpec((D, D), lambda i: (0, 0)),
        ],
        out_specs=pl.BlockSpec((rows_per_block, D), lambda i: (i, 0)),
        out_shape=jax.ShapeDtypeStruct((N, D), jnp.float32),
    )(h, W)


def _mid_body(p_ref, b_ref, w_ref, o_ref):
    h = jnp.maximum(p_ref[0] + p_ref[1] + b_ref[...], 0.0)
    o_ref[...] = jnp.dot(h, w_ref[...], preferred_element_type=jnp.float32)


def _mid(agg, b, W, rows_per_block=2000):
    return pl.pallas_call(
        _mid_body,
        grid=(N // rows_per_block,),
        in_specs=[
            pl.BlockSpec((2, rows_per_block, D), lambda i: (0, i, 0)),
            pl.BlockSpec((1, D), lambda i: (0, 0)),
            pl.BlockSpec((D, D), lambda i: (0, 0)),
        ],
        out_specs=pl.BlockSpec((rows_per_block, D), lambda i: (i, 0)),
        out_shape=jax.ShapeDtypeStruct((N, D), jnp.float32),
    )(agg, b, W)


def _fin_body(p_ref, b_ref, x_ref, o_ref):
    h = jnp.maximum(p_ref[0] + p_ref[1] + b_ref[...], 0.0)
    o_ref[...] = (h + x_ref[...]) * 0.5


def _fin(agg, b, x, rows_per_block=2000):
    return pl.pallas_call(
        _fin_body,
        grid=(N // rows_per_block,),
        in_specs=[
            pl.BlockSpec((2, rows_per_block, D), lambda i: (0, i, 0)),
            pl.BlockSpec((1, D), lambda i: (0, 0)),
            pl.BlockSpec((rows_per_block, D), lambda i: (i, 0)),
        ],
        out_specs=pl.BlockSpec((rows_per_block, D), lambda i: (i, 0)),
        out_shape=jax.ShapeDtypeStruct((N, D), jnp.float32),
    )(agg, b, x)


NBUF = 2


@functools.partial(
    pl.kernel,
    mesh=plsc.VectorSubcoreMesh(core_axis_name="c", subcore_axis_name="s"),
    out_type=jax.ShapeDtypeStruct((2, NROWS, D), jnp.float32),
    scratch_types=[
        pltpu.VMEM((HCPT, CHUNK), jnp.int32),
        pltpu.VMEM((HCPT, CHUNK), jnp.int32),
        pltpu.VMEM((CHUNK, D), jnp.float32),
        pltpu.VMEM((CHUNK, D), jnp.float32),
        pltpu.VMEM_SHARED((NROWS, D), jnp.float32),
        pltpu.SemaphoreType.DMA,
        pltpu.SemaphoreType.DMA,
        pltpu.SemaphoreType.DMA,
        pltpu.SemaphoreType.DMA,
    ],
)
def _sc_aggregate(src_hbm, dst_hbm, hw_hbm, z_hbm, out_hbm,
                  src_v, dst_v, rows0, rows1, agg_sh,
                  gs0, gs1, ss0, ss1):
    bufs = (rows0, rows1)
    gsems = (gs0, gs1)
    ssems = (ss0, ss1)
    cid = lax.axis_index("c")
    sid = lax.axis_index("s")

    row0 = sid * RPT
    pltpu.sync_copy(z_hbm.at[pl.ds(row0, RPT)], agg_sh.at[pl.ds(row0, RPT)])

    base = jnp.where(cid == 0, sid * CPT0, NS * CPT0 + sid * CPT1)
    seg = jnp.where(cid == 0, CPT0 // NSEG, CPT1 // NSEG)
    spairs = seg // NBUF

    for h in range(NSEG):
        off = base + h * seg
        pltpu.sync_copy(src_hbm.at[pl.ds(off, HCPT)], src_v)
        pltpu.sync_copy(dst_hbm.at[pl.ds(off, HCPT)], dst_v)
        if h == 0:
            plsc.subcore_barrier()

        for b in range(NBUF):
            pltpu.async_copy(hw_hbm.at[src_v.at[b]], bufs[b], gsems[b])

        def pair(p, carry):
            for b in range(NBUF):
                k = p * NBUF + b
                pltpu.make_async_copy(hw_hbm.at[src_v.at[k]], bufs[b],
                                      gsems[b]).wait()
                pltpu.async_copy(bufs[b], agg_sh.at[dst_v.at[k]],
                                 ssems[b], add=True)
            for b in range(NBUF):
                kn = p * NBUF + b + NBUF
                pltpu.make_async_copy(bufs[b], agg_sh.at[dst_v.at[0]],
                                      ssems[b]).wait()
                pltpu.async_copy(hw_hbm.at[src_v.at[kn]], bufs[b],
                                 gsems[b])
            return carry

        lax.fori_loop(0, spairs - 1, pair, 0)
        for b in range(NBUF):
            k = (spairs - 1) * NBUF + b
            pltpu.make_async_copy(hw_hbm.at[src_v.at[k]], bufs[b],
                                  gsems[b]).wait()
            pltpu.async_copy(bufs[b], agg_sh.at[dst_v.at[k]], ssems[b],
                             add=True)
        for b in range(NBUF):
            pltpu.make_async_copy(bufs[b], agg_sh.at[dst_v.at[0]],
                                  ssems[b]).wait()

    plsc.subcore_barrier()
    pltpu.sync_copy(agg_sh.at[pl.ds(row0, RPT)],
                    out_hbm.at[cid, pl.ds(row0, RPT)])


def kernel(x, edge_index, W1, b1, W2, b2):
    src = edge_index[0].astype(jnp.int32)
    dst = edge_index[1].astype(jnp.int32)
    pad = E_PAD - E
    src2d = jnp.concatenate([src, jnp.zeros((pad,), jnp.int32)]).reshape(
        TOT_PAD, CHUNK)
    dst2d = jnp.concatenate([dst, jnp.full((pad,), N, jnp.int32)]).reshape(
        TOT_PAD, CHUNK)
    zeros = jnp.zeros((NROWS, D), jnp.float32)
    b1r = b1.reshape(1, D)
    b2r = b2.reshape(1, D)

    hw1 = _matmul(x, W1)
    parts1 = _sc_aggregate(src2d, dst2d, hw1, zeros)
    hw2 = _mid(parts1, b1r, W2)
    parts2 = _sc_aggregate(src2d, dst2d, hw2, zeros)
    return _fin(parts2, b2r, x)


# speedup vs baseline: 1.3768x; 1.0223x over previous
import functools

import jax
import jax.numpy as jnp
from jax import lax
from jax.experimental import pallas as pl
from jax.experimental.pallas import tpu as pltpu
from jax.experimental.pallas import tpu_sc as plsc

N = 10000
E = 320000
D = 128

NC = 2
NS = 16
NW = NC * NS
CHUNK = 128
CPT0 = 112
CPT1 = 48
TOT_CHUNKS = NS * (CPT0 + CPT1)
HCPT = max(CPT0, CPT1) // 2
E_PAD = TOT_CHUNKS * CHUNK
RPT = 632
NROWS = NS * RPT


def _mm_body(h_ref, w_ref, o_ref):
    o_ref[...] = jnp.dot(h_ref[...], w_ref[...],
                         preferred_element_type=jnp.float32)


def _matmul(h, W, rows_per_block=2000):
    return pl.pallas_call(
        _mm_body,
        grid=(N // rows_per_block,),
        in_specs=[
            pl.BlockSpec((rows_per_block, D), lambda i: (i, 0)),
            pl.BlockSpec((D, D), lambda i: (0, 0)),
        ],
        out_specs=pl.BlockSpec((rows_per_block, D), lambda i: (i, 0)),
        out_shape=jax.ShapeDtypeStruct((N, D), jnp.float32),
    )(h, W)


def _mid_body(p_ref, b_ref, w_ref, o_ref):
    h = jnp.maximum(p_ref[0] + p_ref[1] + b_ref[...], 0.0)
    o_ref[...] = jnp.dot(h, w_ref[...], preferred_element_type=jnp.float32)


def _mid(parts, b, W, rows_per_block=2000):
    return pl.pallas_call(
        _mid_body,
        grid=(N // rows_per_block,),
        in_specs=[
            pl.BlockSpec((2, rows_per_block, D), lambda i: (0, i, 0)),
            pl.BlockSpec((1, D), lambda i: (0, 0)),
            pl.BlockSpec((D, D), lambda i: (0, 0)),
        ],
        out_specs=pl.BlockSpec((rows_per_block, D), lambda i: (i, 0)),
        out_shape=jax.ShapeDtypeStruct((N, D), jnp.float32),
    )(parts, b, W)


def _fin_body(p_ref, b_ref, x_ref, o_ref):
    h = jnp.maximum(p_ref[0] + p_ref[1] + b_ref[...], 0.0)
    o_ref[...] = (h + x_ref[...]) * 0.5


def _fin(parts, b, x, rows_per_block=2000):
    return pl.pallas_call(
        _fin_body,
        grid=(N // rows_per_block,),
        in_specs=[
            pl.BlockSpec((2, rows_per_block, D), lambda i: (0, i, 0)),
            pl.BlockSpec((1, D), lambda i: (0, 0)),
            pl.BlockSpec((rows_per_block, D), lambda i: (i, 0)),
        ],
        out_specs=pl.BlockSpec((rows_per_block, D), lambda i: (i, 0)),
        out_shape=jax.ShapeDtypeStruct((N, D), jnp.float32),
    )(parts, b, x)


NBUF = 2


@functools.partial(
    pl.kernel,
    mesh=plsc.VectorSubcoreMesh(core_axis_name="c", subcore_axis_name="s"),
    out_type=jax.ShapeDtypeStruct((2, NROWS, D), jnp.float32),
    scratch_types=[
        pltpu.VMEM((HCPT, CHUNK), jnp.int32),
        pltpu.VMEM((HCPT, CHUNK), jnp.int32),
        pltpu.VMEM((CHUNK, D), jnp.float32),
        pltpu.VMEM((CHUNK, D), jnp.float32),
        pltpu.VMEM_SHARED((NROWS, D), jnp.float32),
        pltpu.SemaphoreType.DMA,
        pltpu.SemaphoreType.DMA,
        pltpu.SemaphoreType.DMA,
        pltpu.SemaphoreType.DMA,
    ],
)
def _sc_aggregate(src_hbm, dst_hbm, hw_hbm, z_hbm, out_hbm,
                  src_v, dst_v, rows0, rows1, agg_sh,
                  gs0, gs1, ss0, ss1):
    bufs = (rows0, rows1)
    gsems = (gs0, gs1)
    ssems = (ss0, ss1)
    cid = lax.axis_index("c")
    sid = lax.axis_index("s")

    row0 = sid * RPT
    pltpu.sync_copy(z_hbm.at[pl.ds(row0, RPT)], agg_sh.at[pl.ds(row0, RPT)])

    base = jnp.where(cid == 0, sid * CPT0, NS * CPT0 + sid * CPT1)
    half = jnp.where(cid == 0, CPT0 // 2, CPT1 // 2)
    hpairs = half // NBUF

    for h in range(2):
        off = base + h * half
        pltpu.sync_copy(src_hbm.at[pl.ds(off, HCPT)], src_v)
        pltpu.sync_copy(dst_hbm.at[pl.ds(off, HCPT)], dst_v)
        if h == 0:
            plsc.subcore_barrier()

        for b in range(NBUF):
            pltpu.async_copy(hw_hbm.at[src_v.at[b]], bufs[b], gsems[b])

        def pair(p, carry):
            for b in range(NBUF):
                k = p * NBUF + b
                pltpu.make_async_copy(hw_hbm.at[src_v.at[k]], bufs[b],
                                      gsems[b]).wait()
                pltpu.async_copy(bufs[b], agg_sh.at[dst_v.at[k]], ssems[b],
                                 add=True)
            for b in range(NBUF):
                kn = p * NBUF + b + NBUF
                pltpu.make_async_copy(bufs[b], agg_sh.at[dst_v.at[0]],
                                      ssems[b]).wait()
                pltpu.async_copy(hw_hbm.at[src_v.at[kn]], bufs[b], gsems[b])
            return carry

        lax.fori_loop(0, hpairs - 1, pair, 0)
        for b in range(NBUF):
            k = (hpairs - 1) * NBUF + b
            pltpu.make_async_copy(hw_hbm.at[src_v.at[k]], bufs[b],
                                  gsems[b]).wait()
            pltpu.async_copy(bufs[b], agg_sh.at[dst_v.at[k]], ssems[b],
                             add=True)
        for b in range(NBUF):
            pltpu.make_async_copy(bufs[b], agg_sh.at[dst_v.at[0]],
                                  ssems[b]).wait()

    plsc.subcore_barrier()
    pltpu.sync_copy(agg_sh.at[pl.ds(row0, RPT)],
                    out_hbm.at[cid, pl.ds(row0, RPT)])


def kernel(x, edge_index, W1, b1, W2, b2):
    src = edge_index[0].astype(jnp.int32)
    dst = edge_index[1].astype(jnp.int32)
    pad = E_PAD - E
    src2d = jnp.concatenate([src, jnp.zeros((pad,), jnp.int32)]).reshape(
        TOT_CHUNKS, CHUNK)
    dst2d = jnp.concatenate([dst, jnp.full((pad,), N, jnp.int32)]).reshape(
        TOT_CHUNKS, CHUNK)
    zeros = jnp.zeros((NROWS, D), jnp.float32)
    b1r = b1.reshape(1, D)
    b2r = b2.reshape(1, D)

    hw1 = _matmul(x, W1)
    parts1 = _sc_aggregate(src2d, dst2d, hw1, zeros)
    hw2 = _mid(parts1, b1r, W2)
    parts2 = _sc_aggregate(src2d, dst2d, hw2, zeros)
    return _fin(parts2, b2r, x)
